# baseline (speedup 1.0000x reference)
import jax
import jax.numpy as jnp
import numpy as np
from jax.experimental import pallas as pl
from jax.experimental.pallas import tpu as pltpu

TB = 2048


def _fused_kernel(xt_ref, w1big_ref, b1e_ref, w2big_ref, b2e_ref,
                  fc1w_ref, fc1b_ref, fc2w_ref, fc2b_ref,
                  fc3w_ref, fc3b_ref,
                  out_ref,
                  pool1_ref, fc1in_ref):
    tb = xt_ref.shape[-1]

    w1 = w1big_ref[...]
    b1 = b1e_ref[...]
    for p in range(14):
        band = xt_ref[2 * p:2 * p + 6, :, :].reshape(192, tb)
        acts = jnp.dot(w1, band, preferred_element_type=jnp.float32
                       ).astype(jnp.bfloat16)
        tpar = jnp.maximum(acts[0:192], acts[192:384])
        tdi = jnp.maximum(tpar[0:96], tpar[96:192])
        pooled = jnp.maximum(tdi + b1, jnp.bfloat16(0))
        pool1_ref[p, :, :, :] = pooled.reshape(6, 16, tb)

    w2 = w2big_ref[...]
    b2 = b2e_ref[...]
    for u in range(5):
        band = pool1_ref[2 * u:2 * u + 6, :, :, :].reshape(576, tb)
        acts = jnp.dot(w2, band, preferred_element_type=jnp.float32
                       ).astype(jnp.bfloat16)
        m01 = jnp.maximum(acts[0:80], acts[80:160])
        m23 = jnp.maximum(acts[160:240], acts[240:320])
        val = jnp.maximum(jnp.maximum(m01, m23) + b2, jnp.bfloat16(0))
        fc1in_ref[80 * u:80 * (u + 1), :] = val

    h1 = jnp.dot(fc1w_ref[...], fc1in_ref[...],
                 preferred_element_type=jnp.float32).astype(jnp.bfloat16)
    h1 = jnp.maximum(h1 + fc1b_ref[...], jnp.bfloat16(0))
    h2 = jnp.dot(fc2w_ref[...], h1,
                 preferred_element_type=jnp.float32).astype(jnp.bfloat16)
    h2 = jnp.maximum(h2 + fc2b_ref[...], jnp.bfloat16(0))
    logits = jnp.dot(fc3w_ref[...], h2, preferred_element_type=jnp.float32)
    out_ref[...] = logits + fc3b_ref[...]


def _expand_weights(w1, b1, wexp, b2):
    kh = np.arange(5)
    i1 = (np.arange(6)[None, None, :] ==
          (np.arange(2)[:, None, None] + kh[None, :, None])).astype(np.float32)
    q = np.arange(16)
    i2 = ((np.arange(32)[None, None, None, :] ==
           (2 * q[None, :, None, None] + np.arange(2)[:, None, None, None]
            + kh[None, None, :, None])) &
          (q[None, :, None, None] < 14)).astype(np.float32)
    t1 = np.einsum("dkr,pqwx->kwpdqrx", i1, i2).reshape(25, 12288)
    w1big = (w1.reshape(6, 25) @ t1).reshape(6, 2, 2, 16, 6, 32)
    w1big = w1big.transpose(1, 2, 0, 3, 4, 5).reshape(384, 192)
    b1e = jnp.where(jnp.arange(16)[None, :] < 14, b1[:, None],
                    0.0).reshape(96, 1)

    w2 = wexp[0:16].reshape(16, 6, 6, 8)[:, :, 0:5, 0:5]
    j2 = (np.arange(16)[None, None, None, :] ==
          (2 * np.arange(5)[None, :, None, None]
           + np.arange(2)[:, None, None, None]
           + kh[None, None, :, None])).astype(np.float32)
    t2 = np.einsum("dkr,jvwx->kwdjvrx", i1, j2).reshape(25, 1920)
    w2big = (w2.reshape(96, 25) @ t2).reshape(16, 6, 2, 2, 5, 6, 16)
    w2big = w2big.transpose(2, 3, 4, 0, 5, 1, 6).reshape(320, 576)
    b2e = jnp.tile(b2, (5, 1))
    return w1big, b1e, w2big, b2e


def _ceil_to(x, m):
    return (x + m - 1) // m * m


def kernel(x, w1, b1, wexp, b2, fc1w, fc1b, fc2w, fc2b, fc3w, fc3b):
    n = x.shape[0]
    num_classes = 10
    npad = _ceil_to(n, TB)
    nb = npad // TB

    x3 = x.reshape(n, 28, 28).astype(jnp.float32)
    if npad != n:
        x3 = jnp.pad(x3, ((0, npad - n), (0, 0), (0, 0)))
    xt = (jnp.pad(x3, ((0, 0), (2, 2), (2, 2)))
          .astype(jnp.bfloat16).transpose(1, 2, 0))

    w1big, b1e, w2big, b2e = _expand_weights(w1, b1, wexp, b2)
    w1big = w1big.astype(jnp.bfloat16)
    w2big = w2big.astype(jnp.bfloat16)
    b1e = b1e.astype(jnp.bfloat16)
    b2e = b2e.astype(jnp.bfloat16)
    fc1wb = fc1w.astype(jnp.bfloat16)
    fc2wb = fc2w.astype(jnp.bfloat16)
    fc3wb = fc3w.astype(jnp.bfloat16)
    fc1bb = fc1b.astype(jnp.bfloat16)
    fc2bb = fc2b.astype(jnp.bfloat16)

    out = pl.pallas_call(
        _fused_kernel,
        out_shape=jax.ShapeDtypeStruct((128, npad), jnp.float32),
        grid=(nb,),
        in_specs=[
            pl.BlockSpec((32, 32, TB), lambda i: (0, 0, i)),
            pl.BlockSpec((384, 192), lambda i: (0, 0)),
            pl.BlockSpec((96, 1), lambda i: (0, 0)),
            pl.BlockSpec((320, 576), lambda i: (0, 0)),
            pl.BlockSpec((80, 1), lambda i: (0, 0)),
            pl.BlockSpec((128, 400), lambda i: (0, 0)),
            pl.BlockSpec((128, 1), lambda i: (0, 0)),
            pl.BlockSpec((128, 128), lambda i: (0, 0)),
            pl.BlockSpec((128, 1), lambda i: (0, 0)),
            pl.BlockSpec((128, 128), lambda i: (0, 0)),
            pl.BlockSpec((128, 1), lambda i: (0, 0)),
        ],
        out_specs=pl.BlockSpec((128, TB), lambda i: (0, i)),
        scratch_shapes=[
            pltpu.VMEM((14, 6, 16, TB), jnp.bfloat16),
            pltpu.VMEM((400, TB), jnp.bfloat16),
        ],
        compiler_params=pltpu.CompilerParams(
            dimension_semantics=("parallel",),
            vmem_limit_bytes=100 * 1024 * 1024),
    )(xt, w1big, b1e, w2big, b2e,
      fc1wb, fc1bb, fc2wb, fc2bb, fc3wb, fc3b)
    return out[:num_classes, :n].T


# speedup vs baseline: 1.0861x; 1.0861x over previous
import jax
import jax.numpy as jnp
import numpy as np
from jax.experimental import pallas as pl
from jax.experimental.pallas import tpu as pltpu

TB = 2048


def _fused_kernel(xt_ref, w1big_ref, b1e_ref, w2big_ref, b2e_ref,
                  fc1w_ref, fc1b_ref, fc2w_ref, fc2b_ref,
                  fc3w_ref, fc3b_ref,
                  out_ref,
                  pool1_ref, fc1in_ref):
    tb = xt_ref.shape[-1]

    w1 = w1big_ref[...]
    b1 = b1e_ref[...]
    for p in range(14):
        band = xt_ref[2 * p:2 * p + 6, :, :].reshape(192, tb)
        acts = jnp.dot(w1, band, preferred_element_type=jnp.float32
                       ).astype(jnp.bfloat16)
        tpar = jnp.maximum(acts[0:192], acts[192:384])
        tdi = jnp.maximum(tpar[0:96], tpar[96:192])
        pooled = jnp.maximum(tdi + b1, jnp.bfloat16(0))
        pool1_ref[p, :, :, :] = pooled.reshape(6, 16, tb)

    w2 = w2big_ref[...]
    b2 = b2e_ref[...]
    for u in range(5):
        band = pool1_ref[2 * u:2 * u + 6, :, :, :].reshape(576, tb)
        acts = jnp.dot(w2, band, preferred_element_type=jnp.float32
                       ).astype(jnp.bfloat16)
        m01 = jnp.maximum(acts[0:80], acts[80:160])
        m23 = jnp.maximum(acts[160:240], acts[240:320])
        val = jnp.maximum(jnp.maximum(m01, m23) + b2, jnp.bfloat16(0))
        fc1in_ref[80 * u:80 * (u + 1), :] = val

    h1 = jnp.dot(fc1w_ref[...], fc1in_ref[...],
                 preferred_element_type=jnp.float32).astype(jnp.bfloat16)
    h1 = jnp.maximum(h1 + fc1b_ref[...], jnp.bfloat16(0))
    h2 = jnp.dot(fc2w_ref[...], h1,
                 preferred_element_type=jnp.float32).astype(jnp.bfloat16)
    h2 = jnp.maximum(h2 + fc2b_ref[...], jnp.bfloat16(0))
    logits = jnp.dot(fc3w_ref[...], h2, preferred_element_type=jnp.float32)
    out_ref[...] = (logits + fc3b_ref[...]).astype(jnp.bfloat16)


def _expand_weights(w1, b1, wexp, b2):
    kh = np.arange(5)
    i1 = (np.arange(6)[None, None, :] ==
          (np.arange(2)[:, None, None] + kh[None, :, None])).astype(np.float32)
    q = np.arange(16)
    i2 = ((np.arange(32)[None, None, None, :] ==
           (2 * q[None, :, None, None] + np.arange(2)[:, None, None, None]
            + kh[None, None, :, None])) &
          (q[None, :, None, None] < 14)).astype(np.float32)
    w1r = w1.reshape(6, 5, 5)
    w1big = jnp.einsum("ckw,dkr,pqwx->pdcqrx", w1r, i1, i2).reshape(384, 192)
    b1e = jnp.where(jnp.arange(16)[None, :] < 14, b1[:, None],
                    0.0).reshape(96, 1)

    w2 = wexp[0:16].reshape(16, 6, 6, 8)[:, :, 0:5, 0:5]
    j2 = (np.arange(16)[None, None, None, :] ==
          (2 * np.arange(5)[None, :, None, None]
           + np.arange(2)[:, None, None, None]
           + kh[None, None, :, None])).astype(np.float32)
    w2big = jnp.einsum("cikw,dkr,jvwx->djvcrix", w2, i1, j2).reshape(320, 576)
    b2e = jnp.tile(b2, (5, 1))
    return w1big, b1e, w2big, b2e


def _ceil_to(x, m):
    return (x + m - 1) // m * m


def kernel(x, w1, b1, wexp, b2, fc1w, fc1b, fc2w, fc2b, fc3w, fc3b):
    n = x.shape[0]
    num_classes = 10
    npad = _ceil_to(n, TB)
    nb = npad // TB

    x3 = x.reshape(n, 28, 28).astype(jnp.float32)
    if npad != n:
        x3 = jnp.pad(x3, ((0, npad - n), (0, 0), (0, 0)))
    xt = (jnp.pad(x3, ((0, 0), (2, 2), (2, 2)))
          .astype(jnp.bfloat16).transpose(1, 2, 0))

    w1big, b1e, w2big, b2e = _expand_weights(w1, b1, wexp, b2)
    w1big = w1big.astype(jnp.bfloat16)
    w2big = w2big.astype(jnp.bfloat16)
    b1e = b1e.astype(jnp.bfloat16)
    b2e = b2e.astype(jnp.bfloat16)
    fc1wb = fc1w.astype(jnp.bfloat16)
    fc2wb = fc2w.astype(jnp.bfloat16)
    fc3wb = fc3w.astype(jnp.bfloat16)
    fc1bb = fc1b.astype(jnp.bfloat16)
    fc2bb = fc2b.astype(jnp.bfloat16)

    out = pl.pallas_call(
        _fused_kernel,
        out_shape=jax.ShapeDtypeStruct((128, npad), jnp.bfloat16),
        grid=(nb,),
        in_specs=[
            pl.BlockSpec((32, 32, TB), lambda i: (0, 0, i)),
            pl.BlockSpec((384, 192), lambda i: (0, 0)),
            pl.BlockSpec((96, 1), lambda i: (0, 0)),
            pl.BlockSpec((320, 576), lambda i: (0, 0)),
            pl.BlockSpec((80, 1), lambda i: (0, 0)),
            pl.BlockSpec((128, 400), lambda i: (0, 0)),
            pl.BlockSpec((128, 1), lambda i: (0, 0)),
            pl.BlockSpec((128, 128), lambda i: (0, 0)),
            pl.BlockSpec((128, 1), lambda i: (0, 0)),
            pl.BlockSpec((128, 128), lambda i: (0, 0)),
            pl.BlockSpec((128, 1), lambda i: (0, 0)),
        ],
        out_specs=pl.BlockSpec((128, TB), lambda i: (0, i)),
        scratch_shapes=[
            pltpu.VMEM((14, 6, 16, TB), jnp.bfloat16),
            pltpu.VMEM((400, TB), jnp.bfloat16),
        ],
        compiler_params=pltpu.CompilerParams(
            dimension_semantics=("parallel",),
            vmem_limit_bytes=64 * 1024 * 1024),
    )(xt, w1big, b1e, w2big, b2e,
      fc1wb, fc1bb, fc2wb, fc2bb, fc3wb, fc3b)
    return out[:num_classes, :n].T.astype(jnp.float32)
